# baseline (speedup 1.0000x reference)
import functools

import jax
import jax.numpy as jnp
from jax.experimental import pallas as pl
from jax.experimental.pallas import tpu as pltpu

_TM = 256
_TW = 1024


def _ff_kernel(x_ref, w1_ref, b1_ref, w2_ref, b2_ref, o_ref, h_ref, *, n_w):
    w = pl.program_id(1)
    h = jnp.dot(x_ref[...], w1_ref[...], preferred_element_type=jnp.float32)
    h = jnp.maximum(h + b1_ref[...], 0.0).astype(jnp.bfloat16)
    h_ref[:, pl.ds(w * _TW, _TW)] = h

    @pl.when(w == n_w - 1)
    def _finish():
        o_ref[...] = (
            jnp.dot(h_ref[...], w2_ref[...], preferred_element_type=jnp.float32)
            + b2_ref[...]
        )


def kernel(x, gate, W1, b1, W2, b2):
    batch, cutoff, dmodel = x.shape
    n_tokens = batch * cutoff
    width = W1.shape[1]

    x2 = x.reshape(n_tokens, dmodel).astype(jnp.bfloat16)
    w1 = W1.astype(jnp.bfloat16)
    w2 = W2.astype(jnp.bfloat16)
    b1f = b1.astype(jnp.float32).reshape(1, width)
    b2f = b2.astype(jnp.float32).reshape(1, dmodel)

    n_m = n_tokens // _TM
    n_w = width // _TW

    out = pl.pallas_call(
        functools.partial(_ff_kernel, n_w=n_w),
        grid=(n_m, n_w),
        in_specs=[
            pl.BlockSpec((_TM, dmodel), lambda m, w: (m, 0)),
            pl.BlockSpec((dmodel, _TW), lambda m, w: (0, w)),
            pl.BlockSpec((1, _TW), lambda m, w: (0, w)),
            pl.BlockSpec((width, dmodel), lambda m, w: (0, 0)),
            pl.BlockSpec((1, dmodel), lambda m, w: (0, 0)),
        ],
        out_specs=pl.BlockSpec((_TM, dmodel), lambda m, w: (m, 0)),
        out_shape=jax.ShapeDtypeStruct((n_tokens, dmodel), jnp.float32),
        scratch_shapes=[pltpu.VMEM((_TM, width), jnp.bfloat16)],
        compiler_params=pltpu.CompilerParams(
            dimension_semantics=("arbitrary", "arbitrary"),
            vmem_limit_bytes=128 * 1024 * 1024,
        ),
    )(x2, w1, b1f, w2, b2f)

    return out.reshape(batch, cutoff, dmodel)
